# baseline (speedup 1.0000x reference)
import functools

import jax
import jax.numpy as jnp
from jax import lax
from jax.experimental import pallas as pl
from jax.experimental.pallas import tpu as pltpu
from jax.experimental.pallas import tpu_sc as plsc

_A = 100000
_F = 2
_B0, _B1 = 4, 4
_T1 = 2
_CD = 16
_BATCH = 16384
_NC, _NS = 2, 16
_NW = _NC * _NS
_PW = _BATCH // _NW
_CH = 64
_NCH = _PW // _CH
_G = 16

_mesh = plsc.VectorSubcoreMesh(
    core_axis_name="c", subcore_axis_name="s", num_cores=_NC, num_subcores=_NS
)


@functools.partial(
    pl.kernel,
    out_type=jax.ShapeDtypeStruct((_BATCH,), jnp.float32),
    mesh=_mesh,
    compiler_params=pltpu.CompilerParams(
        needs_layout_passes=False, use_tc_tiling_on_sc=False
    ),
    scratch_types=[
        pltpu.VMEM((_NCH, _CH), jnp.int32),
        pltpu.VMEM((_PW,), jnp.int32),
        pltpu.VMEM((_PW,), jnp.int32),
        pltpu.VMEM((_PW,), jnp.int32),
        pltpu.VMEM((_CD, _PW), jnp.float32),
        pltpu.VMEM((_NCH, _CD, _CH), jnp.int32),
        pltpu.VMEM((_NCH, _CD, _CH), jnp.float32),
        pltpu.VMEM((_NCH, _CH), jnp.float32),
        pltpu.VMEM((_PW,), jnp.float32),
        pltpu.SemaphoreType.DMA,
        pltpu.SemaphoreType.DMA,
    ],
)
def _sc_gather_dot(a_hbm, st_hbm, bh0_hbm, bh1_hbm, defT_hbm, w_hbm, b_hbm,
                   out_hbm, a_v, st_v, bh0_v, bh1_v, defT_v, idxb_v, val_v,
                   bias_v, out_v, sem, bsem):
    wid = lax.axis_index("s") * _NC + lax.axis_index("c")
    base = wid * _PW

    for c in range(_NCH):
        pltpu.sync_copy(a_hbm.at[pl.ds(base + c * _CH, _CH)], a_v.at[c])
    pltpu.sync_copy(st_hbm.at[pl.ds(base, _PW)], st_v)
    pltpu.sync_copy(bh0_hbm.at[pl.ds(base, _PW)], bh0_v)
    pltpu.sync_copy(bh1_hbm.at[pl.ds(base, _PW)], bh1_v)
    pltpu.sync_copy(defT_hbm.at[:, pl.ds(base, _PW)], defT_v)


    handles = []
    for c in range(_NCH):
        def idx_body(k, _, c=c):
            off = k * _G
            av = a_v[c, pl.ds(off, _G)]
            sv = st_v[pl.ds(c * _CH + off, _G)]
            t1 = bh0_v[pl.ds(c * _CH + off, _G)]
            t2 = bh1_v[pl.ds(c * _CH + off, _G)]
            t1 = jnp.where(t1 >= _F, _F - 1, t1)
            t2 = jnp.where(t2 >= _B0, _B0 - 1, t2)
            fb = ((sv * _T1 + t1) * _B1 + t2) * (_CD * _A) + av
            for j in range(_CD):
                idxb_v[c, j, pl.ds(off, _G)] = fb + j * _A
            return 0
        lax.fori_loop(0, _CH // _G, idx_body, 0)
        chunk = [pltpu.async_copy(w_hbm.at[idxb_v.at[c, j]],
                                  val_v.at[c, j], sem)
                 for j in range(_CD)]
        chunk.append(pltpu.async_copy(b_hbm.at[a_v.at[c]], bias_v.at[c], bsem))
        handles.append(chunk)

    for c in range(_NCH):
        for h in handles[c]:
            h.wait()
        def dot_body(k, _, c=c):
            off = k * _G
            acc = bias_v[c, pl.ds(off, _G)]
            for j in range(_CD):
                rv = val_v[c, j, pl.ds(off, _G)]
                dv = defT_v[j, pl.ds(c * _CH + off, _G)]
                acc = acc + rv * dv
            out_v[pl.ds(c * _CH + off, _G)] = acc
            return 0
        lax.fori_loop(0, _CH // _G, dot_body, 0)

    pltpu.sync_copy(out_v, out_hbm.at[pl.ds(base, _PW)])


def kernel(a, style, bh_pos, def_pos, W, b):
    w1 = jnp.transpose(W[:, :, :_T1], (1, 2, 3, 4, 5, 0)).reshape(-1)
    defT = jnp.transpose(def_pos, (1, 2, 0)).reshape(_CD, _BATCH)
    return _sc_gather_dot(a, style, bh_pos[:, 0], bh_pos[:, 1], defT, w1, b)


# speedup vs baseline: 1.0208x; 1.0208x over previous
import functools

import jax
import jax.numpy as jnp
from jax import lax
from jax.experimental import pallas as pl
from jax.experimental.pallas import tpu as pltpu
from jax.experimental.pallas import tpu_sc as plsc

_A = 100000
_F = 2
_B0, _B1 = 4, 4
_T1 = 2
_CD = 16
_BATCH = 16384
_NC, _NS = 2, 16
_NW = _NC * _NS
_PW = _BATCH // _NW
_CH = 128
_NCH = _PW // _CH
_G = 16

_mesh = plsc.VectorSubcoreMesh(
    core_axis_name="c", subcore_axis_name="s", num_cores=_NC, num_subcores=_NS
)


@functools.partial(
    pl.kernel,
    out_type=jax.ShapeDtypeStruct((_BATCH,), jnp.float32),
    mesh=_mesh,
    compiler_params=pltpu.CompilerParams(
        needs_layout_passes=False, use_tc_tiling_on_sc=False
    ),
    scratch_types=[
        pltpu.VMEM((_NCH, _CH), jnp.int32),
        pltpu.VMEM((_PW,), jnp.int32),
        pltpu.VMEM((2, _PW), jnp.int32),
        pltpu.VMEM((_CD, _PW), jnp.float32),
        pltpu.VMEM((_NCH, _CD, _CH), jnp.int32),
        pltpu.VMEM((_NCH, _CD, _CH), jnp.float32),
        pltpu.VMEM((_NCH, _CH), jnp.float32),
        pltpu.VMEM((_PW,), jnp.float32),
        pltpu.SemaphoreType.DMA,
        pltpu.SemaphoreType.DMA,
    ],
)
def _sc_gather_dot(a_hbm, st_hbm, bhT_hbm, defT_hbm, w_hbm, b_hbm,
                   out_hbm, a_v, st_v, bh2_v, defT_v, idxb_v, val_v,
                   bias_v, out_v, sem, bsem):
    wid = lax.axis_index("s") * _NC + lax.axis_index("c")
    base = wid * _PW

    for c in range(_NCH):
        pltpu.sync_copy(a_hbm.at[pl.ds(base + c * _CH, _CH)], a_v.at[c])
    pltpu.sync_copy(st_hbm.at[pl.ds(base, _PW)], st_v)
    pltpu.sync_copy(bhT_hbm.at[:, pl.ds(base, _PW)], bh2_v)
    pltpu.sync_copy(defT_hbm.at[:, pl.ds(base, _PW)], defT_v)


    handles = []
    for c in range(_NCH):
        def idx_body(k, _, c=c):
            off = k * _G
            av = a_v[c, pl.ds(off, _G)]
            sv = st_v[pl.ds(c * _CH + off, _G)]
            t1 = bh2_v[0, pl.ds(c * _CH + off, _G)]
            t2 = bh2_v[1, pl.ds(c * _CH + off, _G)]
            t1 = jnp.where(t1 >= _F, _F - 1, t1)
            t2 = jnp.where(t2 >= _B0, _B0 - 1, t2)
            fb = ((sv * _T1 + t1) * _B1 + t2) * (_CD * _A) + av
            for j in range(_CD):
                idxb_v[c, j, pl.ds(off, _G)] = fb + j * _A
            return 0
        lax.fori_loop(0, _CH // _G, idx_body, 0)
        chunk = [pltpu.async_copy(w_hbm.at[idxb_v.at[c, j]],
                                  val_v.at[c, j], sem)
                 for j in range(_CD)]
        chunk.append(pltpu.async_copy(b_hbm.at[a_v.at[c]], bias_v.at[c], bsem))
        handles.append(chunk)

    for c in range(_NCH):
        for h in handles[c]:
            h.wait()
        def dot_body(k, _, c=c):
            off = k * _G
            acc = bias_v[c, pl.ds(off, _G)]
            for j in range(_CD):
                rv = val_v[c, j, pl.ds(off, _G)]
                dv = defT_v[j, pl.ds(c * _CH + off, _G)]
                acc = acc + rv * dv
            out_v[pl.ds(c * _CH + off, _G)] = acc
            return 0
        lax.fori_loop(0, _CH // _G, dot_body, 0)

    pltpu.sync_copy(out_v, out_hbm.at[pl.ds(base, _PW)])


def kernel(a, style, bh_pos, def_pos, W, b):
    w1 = jnp.transpose(W[:, :, :_T1], (1, 2, 3, 4, 5, 0)).reshape(-1)
    defT = jnp.transpose(def_pos, (1, 2, 0)).reshape(_CD, _BATCH)
    bhT = jnp.transpose(bh_pos, (1, 0))
    return _sc_gather_dot(a, style, bhT, defT, w1, b)


# speedup vs baseline: 1.0302x; 1.0092x over previous
import functools

import jax
import jax.numpy as jnp
from jax import lax
from jax.experimental import pallas as pl
from jax.experimental.pallas import tpu as pltpu
from jax.experimental.pallas import tpu_sc as plsc

_A = 100000
_F = 2
_B0, _B1 = 4, 4
_T1 = 2
_CD = 16
_BATCH = 16384
_NC, _NS = 2, 16
_NW = _NC * _NS
_PW = _BATCH // _NW
_CH = 128
_NCH = _PW // _CH
_G = 16

_mesh = plsc.VectorSubcoreMesh(
    core_axis_name="c", subcore_axis_name="s", num_cores=_NC, num_subcores=_NS
)


@functools.partial(
    pl.kernel,
    out_type=jax.ShapeDtypeStruct((_BATCH,), jnp.float32),
    mesh=_mesh,
    compiler_params=pltpu.CompilerParams(
        needs_layout_passes=False, use_tc_tiling_on_sc=False
    ),
    scratch_types=[
        pltpu.VMEM((_NCH, _CH), jnp.int32),
        pltpu.VMEM((_PW,), jnp.int32),
        pltpu.VMEM((2, _PW), jnp.int32),
        pltpu.VMEM((_CD, _PW), jnp.float32),
        pltpu.VMEM((_NCH, _CD, _CH), jnp.int32),
        pltpu.VMEM((_NCH, _CD, _CH), jnp.float32),
        pltpu.VMEM((_NCH, _CH), jnp.float32),
        pltpu.VMEM((_PW,), jnp.float32),
        pltpu.SemaphoreType.DMA,
        pltpu.SemaphoreType.DMA,
    ],
)
def _sc_gather_dot(a_hbm, st_hbm, bhT_hbm, defT_hbm, w_hbm, b_hbm,
                   out_hbm, a_v, st_v, bh2_v, defT_v, idxb_v, val_v,
                   bias_v, out_v, sem, bsem):
    wid = lax.axis_index("s") * _NC + lax.axis_index("c")
    base = wid * _PW

    stage = [pltpu.async_copy(a_hbm.at[pl.ds(base + c * _CH, _CH)],
                              a_v.at[c], bsem) for c in range(_NCH)]
    stage.append(pltpu.async_copy(st_hbm.at[pl.ds(base, _PW)], st_v, bsem))
    stage.append(pltpu.async_copy(bhT_hbm.at[:, pl.ds(base, _PW)], bh2_v, bsem))
    stage.append(pltpu.async_copy(defT_hbm.at[:, pl.ds(base, _PW)], defT_v, bsem))
    for h in stage:
        h.wait()


    handles = []
    for c in range(_NCH):
        def idx_body(k, _, c=c):
            off = k * _G
            av = a_v[c, pl.ds(off, _G)]
            sv = st_v[pl.ds(c * _CH + off, _G)]
            t1 = bh2_v[0, pl.ds(c * _CH + off, _G)]
            t2 = bh2_v[1, pl.ds(c * _CH + off, _G)]
            t1 = jnp.where(t1 >= _F, _F - 1, t1)
            t2 = jnp.where(t2 >= _B0, _B0 - 1, t2)
            fb = ((sv * _T1 + t1) * _B1 + t2) * (_CD * _A) + av
            for j in range(_CD):
                idxb_v[c, j, pl.ds(off, _G)] = fb + j * _A
            return 0
        lax.fori_loop(0, _CH // _G, idx_body, 0)
        chunk = [pltpu.async_copy(w_hbm.at[idxb_v.at[c, j]],
                                  val_v.at[c, j], sem)
                 for j in range(_CD)]
        chunk.append(pltpu.async_copy(b_hbm.at[a_v.at[c]], bias_v.at[c], bsem))
        handles.append(chunk)

    for c in range(_NCH):
        for h in handles[c]:
            h.wait()
        def dot_body(k, _, c=c):
            off = k * _G
            acc = bias_v[c, pl.ds(off, _G)]
            for j in range(_CD):
                rv = val_v[c, j, pl.ds(off, _G)]
                dv = defT_v[j, pl.ds(c * _CH + off, _G)]
                acc = acc + rv * dv
            out_v[pl.ds(c * _CH + off, _G)] = acc
            return 0
        lax.fori_loop(0, _CH // _G, dot_body, 0)

    pltpu.sync_copy(out_v, out_hbm.at[pl.ds(base, _PW)])


def kernel(a, style, bh_pos, def_pos, W, b):
    w1 = jnp.transpose(W[:, :, :_T1], (1, 2, 3, 4, 5, 0)).reshape(-1)
    defT = jnp.transpose(def_pos, (1, 2, 0)).reshape(_CD, _BATCH)
    bhT = jnp.transpose(bh_pos, (1, 0))
    return _sc_gather_dot(a, style, bhT, defT, w1, b)


# speedup vs baseline: 1.0323x; 1.0021x over previous
import functools

import jax
import jax.numpy as jnp
from jax import lax
from jax.experimental import pallas as pl
from jax.experimental.pallas import tpu as pltpu
from jax.experimental.pallas import tpu_sc as plsc

_A = 100000
_F = 2
_B0, _B1 = 4, 4
_T1 = 2
_CD = 16
_BATCH = 16384
_NC, _NS = 2, 16
_NW = _NC * _NS
_PW = _BATCH // _NW
_CH = 128
_NCH = _PW // _CH
_G = 16

_mesh = plsc.VectorSubcoreMesh(
    core_axis_name="c", subcore_axis_name="s", num_cores=_NC, num_subcores=_NS
)


@functools.partial(
    pl.kernel,
    out_type=jax.ShapeDtypeStruct((_BATCH,), jnp.float32),
    mesh=_mesh,
    compiler_params=pltpu.CompilerParams(
        needs_layout_passes=False, use_tc_tiling_on_sc=False
    ),
    scratch_types=[
        pltpu.VMEM((_NCH, _CH), jnp.int32),
        pltpu.VMEM((_PW,), jnp.int32),
        pltpu.VMEM((2, _PW), jnp.int32),
        pltpu.VMEM((_CD, _PW), jnp.float32),
        pltpu.VMEM((_NCH, _CD, _CH), jnp.int32),
        pltpu.VMEM((_NCH, _CD, _CH), jnp.float32),
        pltpu.VMEM((_NCH, _CH), jnp.float32),
        pltpu.VMEM((_PW,), jnp.float32),
        pltpu.SemaphoreType.DMA,
        pltpu.SemaphoreType.DMA,
        pltpu.SemaphoreType.DMA,
    ],
)
def _sc_gather_dot(a_hbm, st_hbm, bhT_hbm, defT_hbm, w_hbm, b_hbm,
                   out_hbm, a_v, st_v, bh2_v, defT_v, idxb_v, val_v,
                   bias_v, out_v, sem, bsem, dsem):
    wid = lax.axis_index("s") * _NC + lax.axis_index("c")
    base = wid * _PW

    stage = [pltpu.async_copy(a_hbm.at[pl.ds(base + c * _CH, _CH)],
                              a_v.at[c], bsem) for c in range(_NCH)]
    stage.append(pltpu.async_copy(st_hbm.at[pl.ds(base, _PW)], st_v, bsem))
    stage.append(pltpu.async_copy(bhT_hbm.at[:, pl.ds(base, _PW)], bh2_v, bsem))
    dstage = pltpu.async_copy(defT_hbm.at[:, pl.ds(base, _PW)], defT_v, dsem)
    for h in stage:
        h.wait()


    handles = []
    for c in range(_NCH):
        def idx_body(k, _, c=c):
            off = k * _G
            av = a_v[c, pl.ds(off, _G)]
            sv = st_v[pl.ds(c * _CH + off, _G)]
            t1 = bh2_v[0, pl.ds(c * _CH + off, _G)]
            t2 = bh2_v[1, pl.ds(c * _CH + off, _G)]
            t1 = jnp.where(t1 >= _F, _F - 1, t1)
            t2 = jnp.where(t2 >= _B0, _B0 - 1, t2)
            fb = ((sv * _T1 + t1) * _B1 + t2) * (_CD * _A) + av
            for j in range(_CD):
                idxb_v[c, j, pl.ds(off, _G)] = fb + j * _A
            return 0
        lax.fori_loop(0, _CH // _G, idx_body, 0)
        chunk = [pltpu.async_copy(w_hbm.at[idxb_v.at[c, j]],
                                  val_v.at[c, j], sem)
                 for j in range(_CD)]
        chunk.append(pltpu.async_copy(b_hbm.at[a_v.at[c]], bias_v.at[c], bsem))
        handles.append(chunk)

    dstage.wait()
    for c in range(_NCH):
        for h in handles[c]:
            h.wait()
        def dot_body(k, _, c=c):
            off = k * _G
            acc = bias_v[c, pl.ds(off, _G)]
            for j in range(_CD):
                rv = val_v[c, j, pl.ds(off, _G)]
                dv = defT_v[j, pl.ds(c * _CH + off, _G)]
                acc = acc + rv * dv
            out_v[pl.ds(c * _CH + off, _G)] = acc
            return 0
        lax.fori_loop(0, _CH // _G, dot_body, 0)

    pltpu.sync_copy(out_v, out_hbm.at[pl.ds(base, _PW)])


def kernel(a, style, bh_pos, def_pos, W, b):
    w1 = jnp.transpose(W[:, :, :_T1], (1, 2, 3, 4, 5, 0)).reshape(-1)
    defT = jnp.transpose(def_pos, (1, 2, 0)).reshape(_CD, _BATCH)
    bhT = jnp.transpose(bh_pos, (1, 0))
    return _sc_gather_dot(a, style, bhT, defT, w1, b)


# speedup vs baseline: 1.0324x; 1.0001x over previous
import functools

import jax
import jax.numpy as jnp
from jax import lax
from jax.experimental import pallas as pl
from jax.experimental.pallas import tpu as pltpu
from jax.experimental.pallas import tpu_sc as plsc

_A = 100000
_F = 2
_B0, _B1 = 4, 4
_T1 = 2
_CD = 16
_BATCH = 16384
_NC, _NS = 2, 16
_NW = _NC * _NS
_PW = _BATCH // _NW
_CH = 128
_NCH = _PW // _CH
_G = 16

_mesh = plsc.VectorSubcoreMesh(
    core_axis_name="c", subcore_axis_name="s", num_cores=_NC, num_subcores=_NS
)


@functools.partial(
    pl.kernel,
    out_type=jax.ShapeDtypeStruct((_BATCH,), jnp.float32),
    mesh=_mesh,
    compiler_params=pltpu.CompilerParams(
        needs_layout_passes=False, use_tc_tiling_on_sc=False
    ),
    scratch_types=[
        pltpu.VMEM((_NCH, _CH), jnp.int32),
        pltpu.VMEM((_PW,), jnp.int32),
        pltpu.VMEM((2, _PW), jnp.int32),
        pltpu.VMEM((_CD, _PW), jnp.float32),
        pltpu.VMEM((_NCH, _CD, _CH), jnp.int32),
        pltpu.VMEM((_NCH, _CD, _CH), jnp.float32),
        pltpu.VMEM((_NCH, _CH), jnp.float32),
        pltpu.VMEM((_PW,), jnp.float32),
        pltpu.SemaphoreType.DMA,
        pltpu.SemaphoreType.DMA,
        pltpu.SemaphoreType.DMA,
    ],
)
def _sc_gather_dot(a_hbm, st_hbm, bhT_hbm, defT_hbm, w_hbm, b_hbm,
                   out_hbm, a_v, st_v, bh2_v, defT_v, idxb_v, val_v,
                   bias_v, out_v, sem, bsem, dsem):
    wid = lax.axis_index("s") * _NC + lax.axis_index("c")
    base = wid * _PW

    stage = [pltpu.async_copy(a_hbm.at[pl.ds(base + c * _CH, _CH)],
                              a_v.at[c], bsem) for c in range(_NCH)]
    stage.append(pltpu.async_copy(st_hbm.at[pl.ds(base, _PW)], st_v, bsem))
    stage.append(pltpu.async_copy(bhT_hbm.at[:, pl.ds(base, _PW)], bh2_v, bsem))
    dstage = pltpu.async_copy(defT_hbm.at[:, pl.ds(base, _PW)], defT_v, dsem)
    for h in stage:
        h.wait()


    handles = []
    for c in range(_NCH):
        def idx_body(k, _, c=c):
            off = k * _G
            av = a_v[c, pl.ds(off, _G)]
            sv = st_v[pl.ds(c * _CH + off, _G)]
            t1 = bh2_v[0, pl.ds(c * _CH + off, _G)]
            t2 = bh2_v[1, pl.ds(c * _CH + off, _G)]
            t1 = jnp.where(t1 >= _F, _F - 1, t1)
            t2 = jnp.where(t2 >= _B0, _B0 - 1, t2)
            fb = ((sv * _T1 + t1) * _B1 + t2) * (_CD * _A) + av
            for j in range(_CD):
                idxb_v[c, j, pl.ds(off, _G)] = fb + j * _A
            return 0
        lax.fori_loop(0, _CH // _G, idx_body, 0)
        chunk = [pltpu.async_copy(w_hbm.at[idxb_v.at[c, j]],
                                  val_v.at[c, j], sem)
                 for j in range(_CD)]
        chunk.append(pltpu.async_copy(b_hbm.at[a_v.at[c]], bias_v.at[c], bsem))
        handles.append(chunk)

    dstage.wait()
    for chunk in handles:
        for h in chunk:
            h.wait()

    for c in range(_NCH):
        def dot_body(k, _, c=c):
            off = k * _G
            acc = bias_v[c, pl.ds(off, _G)]
            for j in range(_CD):
                rv = val_v[c, j, pl.ds(off, _G)]
                dv = defT_v[j, pl.ds(c * _CH + off, _G)]
                acc = acc + rv * dv
            out_v[pl.ds(c * _CH + off, _G)] = acc
            return 0
        lax.fori_loop(0, _CH // _G, dot_body, 0)

    pltpu.sync_copy(out_v, out_hbm.at[pl.ds(base, _PW)])


def kernel(a, style, bh_pos, def_pos, W, b):
    w1 = jnp.transpose(W[:, :, :_T1], (1, 2, 3, 4, 5, 0)).reshape(-1)
    defT = jnp.transpose(def_pos, (1, 2, 0)).reshape(_CD, _BATCH)
    bhT = jnp.transpose(bh_pos, (1, 0))
    return _sc_gather_dot(a, style, bhT, defT, w1, b)
